# baseline (speedup 1.0000x reference)
import jax
import jax.numpy as jnp
from jax import lax
from jax.experimental import pallas as pl
from jax.experimental.pallas import tpu as pltpu
from jax.experimental.pallas import tpu_sc as plsc

NUM_EMB = 100000
EMB_DIM = 300
BATCH = 1024
SEQ = 50
B = BATCH * SEQ

_info = plsc.get_sparse_core_info()
NC = _info.num_cores
NS = _info.num_subcores
NW = NC * NS
L = _info.num_lanes
ROWS_PER = -(-EMB_DIM // NW)
REM = EMB_DIM % NW
SCC = 5
JC = SCC * BATCH
NJC = B // JC
UNROLL = 16


def _gather_kernel(idx_hbm, tab_hbm, out_hbm, idx_v, row_v, out_v,
                   rsem, isem0, isem1, osem0, osem1):
    wid = lax.axis_index("s") * NC + lax.axis_index("c")
    isems = (isem0, isem1)
    osems = (osem0, osem1)
    nrows = jnp.where(wid < REM, ROWS_PER, ROWS_PER - 1)

    def row_body(t, carry):
        r = wid + t * NW
        rcp = pltpu.async_copy(tab_hbm.at[r], row_v, rsem)
        icp = pltpu.async_copy(idx_hbm.at[pl.ds(0, JC)], idx_v.at[0], isems[0])
        rcp.wait()
        icp.wait()
        pending_w = [None, None]
        pending_i = [None, None]
        for c in range(NJC):
            slot = c % 2
            if c + 1 < NJC:
                pending_i[1 - slot] = pltpu.async_copy(
                    idx_hbm.at[pl.ds((c + 1) * JC, JC)],
                    idx_v.at[1 - slot], isems[1 - slot])
            if pending_w[slot] is not None:
                pending_w[slot].wait()

            @plsc.parallel_loop(0, JC, L, unroll=UNROLL)
            def _gather(o, slot=slot):
                si = o >> 10
                ob = o & (BATCH - 1)
                iv = idx_v[slot, pl.ds(o, L)]
                out_v[slot, si, pl.ds(ob, L)] = plsc.load_gather(row_v, [iv])
            pending_w[slot] = pltpu.async_copy(
                out_v.at[slot], out_hbm.at[pl.ds(c * SCC, SCC), r],
                osems[slot])
            if c + 1 < NJC:
                pending_i[1 - slot].wait()
        for w in pending_w:
            if w is not None:
                w.wait()
        return carry

    lax.fori_loop(0, nrows, row_body, 0)


def kernel(x, table):
    idx = x.T.reshape(B).astype(jnp.int32)
    tab_t = table.T
    mesh = plsc.VectorSubcoreMesh(core_axis_name="c", subcore_axis_name="s")
    out_sdb = pl.kernel(
        _gather_kernel,
        out_type=jax.ShapeDtypeStruct((SEQ, EMB_DIM, BATCH), jnp.float32),
        mesh=mesh,
        scratch_types=[
            pltpu.VMEM((2, JC), jnp.int32),
            pltpu.VMEM((NUM_EMB,), jnp.float32),
            pltpu.VMEM((2, SCC, BATCH), jnp.float32),
            pltpu.SemaphoreType.DMA,
            pltpu.SemaphoreType.DMA,
            pltpu.SemaphoreType.DMA,
            pltpu.SemaphoreType.DMA,
            pltpu.SemaphoreType.DMA,
        ],
        compiler_params=pltpu.CompilerParams(needs_layout_passes=False),
    )(idx, tab_t)
    return out_sdb.transpose(2, 0, 1)


# speedup vs baseline: 1.0462x; 1.0462x over previous
import jax
import jax.numpy as jnp
from jax import lax
from jax.experimental import pallas as pl
from jax.experimental.pallas import tpu as pltpu
from jax.experimental.pallas import tpu_sc as plsc

NUM_EMB = 100000
EMB_DIM = 300
BATCH = 1024
SEQ = 50
B = BATCH * SEQ

_info = plsc.get_sparse_core_info()
NC = _info.num_cores
NS = _info.num_subcores
NW = NC * NS
L = _info.num_lanes
ROWS_PER = -(-EMB_DIM // NW)
REM = EMB_DIM % NW
SCC = 5
JC = SCC * BATCH
NJC = B // JC
UNROLL = 8
NBUF = 2
NOBUF = 5


def _gather_kernel(idx_hbm, tab_hbm, out_hbm, idx_v, row_v, out_v,
                   rsem, isem0, isem1, osem0, osem1, osem2, osem3, osem4):
    wid = lax.axis_index("s") * NC + lax.axis_index("c")
    isems = (isem0, isem1)
    osems = (osem0, osem1, osem2, osem3, osem4)
    nrows = jnp.where(wid < REM, ROWS_PER, ROWS_PER - 1)

    def row_body(t, carry):
        r = wid + t * NW
        rcp = pltpu.async_copy(tab_hbm.at[r], row_v, rsem)
        icps = {}
        for c in range(NBUF):
            icps[c] = pltpu.async_copy(
                idx_hbm.at[pl.ds(c * JC, JC)], idx_v.at[c], isems[c])
        rcp.wait()
        ocps = {}
        for m in range(SEQ):
            c = m // SCC
            islot = c % NBUF
            oslot = m % NOBUF
            if m % SCC == 0:
                icps[c].wait()
            if m - NOBUF >= 0:
                ocps[m - NOBUF].wait()

            @plsc.parallel_loop(0, BATCH, L, unroll=UNROLL)
            def _gather(o, m=m, islot=islot, oslot=oslot):
                iv = idx_v[islot, pl.ds((m % SCC) * BATCH + o, L)]
                out_v[oslot, 0, pl.ds(o, L)] = plsc.load_gather(row_v, [iv])
            ocps[m] = pltpu.async_copy(
                out_v.at[oslot], out_hbm.at[pl.ds(m, 1), r], osems[oslot])
            if m % SCC == SCC - 1 and c + NBUF < NJC:
                icps[c + NBUF] = pltpu.async_copy(
                    idx_hbm.at[pl.ds((c + NBUF) * JC, JC)],
                    idx_v.at[islot], isems[islot])
        for m in range(SEQ - NOBUF, SEQ):
            ocps[m].wait()
        return carry

    lax.fori_loop(0, nrows, row_body, 0)


def kernel(x, table):
    idx = x.T.reshape(B).astype(jnp.int32)
    tab_t = table.T
    mesh = plsc.VectorSubcoreMesh(core_axis_name="c", subcore_axis_name="s")
    out_sdb = pl.kernel(
        _gather_kernel,
        out_type=jax.ShapeDtypeStruct((SEQ, EMB_DIM, BATCH), jnp.float32),
        mesh=mesh,
        scratch_types=[
            pltpu.VMEM((NBUF, JC), jnp.int32),
            pltpu.VMEM((NUM_EMB,), jnp.float32),
            pltpu.VMEM((NOBUF, 1, BATCH), jnp.float32),
            pltpu.SemaphoreType.DMA,
            pltpu.SemaphoreType.DMA,
            pltpu.SemaphoreType.DMA,
            pltpu.SemaphoreType.DMA,
            pltpu.SemaphoreType.DMA,
            pltpu.SemaphoreType.DMA,
            pltpu.SemaphoreType.DMA,
            pltpu.SemaphoreType.DMA,
        ],
        compiler_params=pltpu.CompilerParams(needs_layout_passes=False),
    )(idx, tab_t)
    return out_sdb.transpose(2, 0, 1)
